# Initial kernel scaffold; baseline (speedup 1.0000x reference)
import jax
import jax.numpy as jnp
from jax.experimental import pallas as pl


def kernel(inputs, channel_embeddings, positional_embeddings):
    raise NotImplementedError("write your pallas kernel here")


# speedup vs baseline: 3.9892x; 3.9892x over previous
import jax
import jax.numpy as jnp
from jax.experimental import pallas as pl


def _body(x_ref, ce_ref, pos_ref, out_ref):
    _, T, rows, C = x_ref.shape
    _, hs, W, _, D = out_ref.shape
    ce = ce_ref[...]
    pos = pos_ref[...]
    for t in range(T):
        y = jnp.dot(x_ref[0, t], ce, preferred_element_type=jnp.float32)
        out_ref[0, :, :, t, :] = (y + pos).reshape(hs, W, D)


@jax.jit
def kernel(inputs, channel_embeddings, positional_embeddings):
    B, T, H, W, C = inputs.shape
    _, D = channel_embeddings.shape
    HW = H * W
    hs = 8
    nh = H // hs

    x = inputs.reshape(B, T, HW, C)
    pos = positional_embeddings.reshape(HW, D)

    out = pl.pallas_call(
        _body,
        grid=(B, nh),
        in_specs=[
            pl.BlockSpec((1, T, hs * W, C), lambda b, h: (b, 0, h, 0)),
            pl.BlockSpec((C, D), lambda b, h: (0, 0)),
            pl.BlockSpec((hs * W, D), lambda b, h: (h, 0)),
        ],
        out_specs=pl.BlockSpec((1, hs, W, T, D), lambda b, h: (b, h, 0, 0, 0)),
        out_shape=jax.ShapeDtypeStruct((B, H, W, T, D), jnp.float32),
    )(x, channel_embeddings, pos)

    return out
